# Initial kernel scaffold; baseline (speedup 1.0000x reference)
import jax
import jax.numpy as jnp
from jax.experimental import pallas as pl


def kernel(feat, num_graphs):
    raise NotImplementedError("write your pallas kernel here")


# speedup vs baseline: 4.2035x; 4.2035x over previous
import functools

import jax
import jax.numpy as jnp
from jax import lax
from jax.experimental import pallas as pl
from jax.experimental.pallas import tpu as pltpu

B = 256
K = 64


def _rowmax_kernel(x_ref, o_ref):
    o_ref[...] = jnp.max(x_ref[...], axis=1, keepdims=True)


def _topk_kernel(keys_ref, idx_ref):
    keys = keys_ref[...]
    nb, n = keys.shape
    iota = lax.broadcasted_iota(jnp.int32, (nb, n), 1)
    neg = jnp.float32(-jnp.inf)
    for k in range(K):
        m = jnp.max(keys, axis=1, keepdims=True)
        cand = jnp.where(keys == m, iota, n)
        a = jnp.min(cand, axis=1)
        idx_ref[:, k] = a
        keys = jnp.where(iota == a[:, None], neg, keys)


def _bitonic_sort_rows(x):
    r, d = x.shape
    iota = lax.broadcasted_iota(jnp.int32, (r, d), 1)
    k = 2
    while k <= d:
        j = k // 2
        while j >= 1:
            up = pltpu.roll(x, d - j, axis=1)
            dn = pltpu.roll(x, j, axis=1)
            lower = (iota & j) == 0
            partner = jnp.where(lower, up, dn)
            asc = (iota & k) == 0
            keep_min = lower == asc
            x = jnp.where(keep_min, jnp.minimum(x, partner),
                          jnp.maximum(x, partner))
            j //= 2
        k *= 2
    return x


def _gather_sort_kernel(idx_ref, feat_ref, o_ref, scratch):
    g = pl.program_id(0)
    for k in range(K):
        node = idx_ref[g, k]
        scratch[k:k + 1, :] = feat_ref[pl.ds(node, 1), :]
    o_ref[0] = _bitonic_sort_rows(scratch[...])


def kernel(feat, num_graphs):
    ntot, d = feat.shape
    n = ntot // B
    rb = 2048

    maxes = pl.pallas_call(
        _rowmax_kernel,
        grid=(ntot // rb,),
        in_specs=[pl.BlockSpec((rb, d), lambda i: (i, 0))],
        out_specs=pl.BlockSpec((rb, 1), lambda i: (i, 0)),
        out_shape=jax.ShapeDtypeStruct((ntot, 1), jnp.float32),
    )(feat)
    keys = maxes.reshape(B, n)

    idx = pl.pallas_call(
        _topk_kernel,
        out_shape=jax.ShapeDtypeStruct((B, K), jnp.int32),
    )(keys)

    grid_spec = pltpu.PrefetchScalarGridSpec(
        num_scalar_prefetch=1,
        grid=(B,),
        in_specs=[pl.BlockSpec((n, d), lambda g, idx_ref: (g, 0))],
        out_specs=pl.BlockSpec((1, K, d), lambda g, idx_ref: (g, 0, 0)),
        scratch_shapes=[pltpu.VMEM((K, d), jnp.float32)],
    )
    out = pl.pallas_call(
        _gather_sort_kernel,
        grid_spec=grid_spec,
        out_shape=jax.ShapeDtypeStruct((B, K, d), jnp.float32),
    )(idx, feat)

    return out.reshape(B, K * d)


# speedup vs baseline: 9.4690x; 2.2526x over previous
import functools

import jax
import jax.numpy as jnp
from jax import lax
from jax.experimental import pallas as pl
from jax.experimental.pallas import tpu as pltpu

B = 256
K = 64


G = 8


def _rowmax_kernel(x_ref, o_ref):
    rb, d = x_ref.shape
    gb = o_ref.shape[0]
    o_ref[...] = jnp.max(x_ref[...].reshape(gb, rb // gb, d), axis=2)


def _topk_kernel(keys_ref, idx_ref):
    keys = keys_ref[...]
    nb, n = keys.shape
    iota = lax.broadcasted_iota(jnp.int32, (nb, n), 1)
    neg = jnp.float32(-jnp.inf)
    for k in range(K):
        m = jnp.max(keys, axis=1, keepdims=True)
        cand = jnp.where(keys == m, iota, n)
        a = jnp.min(cand, axis=1)
        idx_ref[:, k] = a
        keys = jnp.where(iota == a[:, None], neg, keys)


def _bitonic_sort_rows(x):
    r, d = x.shape
    iota = lax.broadcasted_iota(jnp.int32, (r, d), 1)
    k = 2
    while k <= d:
        j = k // 2
        while j >= 1:
            up = pltpu.roll(x, d - j, axis=1)
            dn = pltpu.roll(x, j, axis=1)
            lower = (iota & j) == 0
            partner = jnp.where(lower, up, dn)
            asc = (iota & k) == 0
            keep_min = lower == asc
            x = jnp.where(keep_min, jnp.minimum(x, partner),
                          jnp.maximum(x, partner))
            j //= 2
        k *= 2
    return x


def _gather_sort_kernel(idx_ref, feat_ref, o_ref):
    n = feat_ref.shape[0] // G
    rows = []
    for g in range(G):
        idx_g = idx_ref[g:g + 1, :]
        pt = (lax.broadcasted_iota(jnp.int32, (n, K), 0) == idx_g)
        rows.append(lax.dot_general(
            pt.astype(jnp.float32), feat_ref[g * n:(g + 1) * n, :],
            (((0,), (0,)), ((), ())),
            precision=lax.Precision.HIGHEST,
            preferred_element_type=jnp.float32))
    x = jnp.concatenate(rows, axis=0)
    o_ref[...] = _bitonic_sort_rows(x)


def kernel(feat, num_graphs):
    ntot, d = feat.shape
    n = ntot // B
    rb = 4096

    gb = rb // n

    keys = pl.pallas_call(
        _rowmax_kernel,
        grid=(ntot // rb,),
        in_specs=[pl.BlockSpec((rb, d), lambda i: (i, 0))],
        out_specs=pl.BlockSpec((gb, n), lambda i: (i, 0)),
        out_shape=jax.ShapeDtypeStruct((B, n), jnp.float32),
    )(feat)

    idx = pl.pallas_call(
        _topk_kernel,
        out_shape=jax.ShapeDtypeStruct((B, K), jnp.int32),
    )(keys)

    out = pl.pallas_call(
        _gather_sort_kernel,
        grid=(B // G,),
        in_specs=[
            pl.BlockSpec((G, K), lambda s: (s, 0)),
            pl.BlockSpec((G * n, d), lambda s: (s, 0)),
        ],
        out_specs=pl.BlockSpec((G * K, d), lambda s: (s, 0)),
        out_shape=jax.ShapeDtypeStruct((B * K, d), jnp.float32),
    )(idx, feat)

    return out.reshape(B, K * d)


# speedup vs baseline: 9.9154x; 1.0471x over previous
import functools

import jax
import jax.numpy as jnp
from jax import lax
from jax.experimental import pallas as pl
from jax.experimental.pallas import tpu as pltpu
from jax.experimental.pallas import tpu_sc as plsc

B = 256
K = 64
G = 8


def _rowmax_kernel(x_ref, o_ref):
    rb, d = x_ref.shape
    gb = o_ref.shape[0]
    o_ref[...] = jnp.max(x_ref[...].reshape(gb, rb // gb, d), axis=2)


def _topk_kernel(keys_ref, idx_ref):
    keys = keys_ref[...]
    nb, n = keys.shape
    iota = lax.broadcasted_iota(jnp.int32, (nb, n), 1)
    base = lax.broadcasted_iota(jnp.int32, (nb,), 0) * n
    neg = jnp.float32(-jnp.inf)
    for k in range(K):
        m = jnp.max(keys, axis=1, keepdims=True)
        cand = jnp.where(keys == m, iota, n)
        a = jnp.min(cand, axis=1)
        idx_ref[:, k] = base + a
        keys = jnp.where(iota == a[:, None], neg, keys)


def _bitonic_sort_rows(x):
    r, d = x.shape
    iota = lax.broadcasted_iota(jnp.int32, (r, d), 1)
    k = 2
    while k <= d:
        j = k // 2
        while j >= 1:
            up = pltpu.roll(x, d - j, axis=1)
            dn = pltpu.roll(x, j, axis=1)
            lower = (iota & j) == 0
            partner = jnp.where(lower, up, dn)
            asc = (iota & k) == 0
            keep_min = lower == asc
            x = jnp.where(keep_min, jnp.minimum(x, partner),
                          jnp.maximum(x, partner))
            j //= 2
        k *= 2
    return x


def _sort_kernel(x_ref, o_ref):
    o_ref[...] = _bitonic_sort_rows(x_ref[...])


def _make_sc_gather(ntot, d):
    nsel = B * K
    nw = 32
    bpw = nsel // nw
    ch = 128
    nch = bpw // ch
    mesh = plsc.VectorSubcoreMesh(core_axis_name="c", subcore_axis_name="s")

    @functools.partial(
        pl.kernel, mesh=mesh,
        out_type=jax.ShapeDtypeStruct((nsel, d), jnp.float32),
        scratch_types=[
            pltpu.VMEM((bpw,), jnp.int32),
            pltpu.VMEM((ch, d), jnp.float32),
            pltpu.VMEM((ch, d), jnp.float32),
            pltpu.SemaphoreType.DMA,
            pltpu.SemaphoreType.DMA,
            pltpu.SemaphoreType.DMA,
            pltpu.SemaphoreType.DMA,
        ],
    )
    def sc_gather(feat_hbm, idx_hbm, out_hbm, idx_v, buf0, buf1,
                  sem0, sem1, osem0, osem1):
        wid = lax.axis_index("s") * 2 + lax.axis_index("c")
        base = wid * bpw
        pltpu.sync_copy(idx_hbm.at[pl.ds(base, bpw)], idx_v)
        bufs, sems = (buf0, buf1), (sem0, sem1)
        osems = (osem0, osem1)
        gathers = [None, None]
        outs = [None, None]
        gathers[0] = pltpu.async_copy(
            feat_hbm.at[idx_v.at[pl.ds(0, ch)]], bufs[0], sems[0])
        for c in range(nch):
            b = c % 2
            o = (c + 1) % 2
            gathers[b].wait()
            if c + 1 < nch:
                if outs[o] is not None:
                    outs[o].wait()
                gathers[o] = pltpu.async_copy(
                    feat_hbm.at[idx_v.at[pl.ds((c + 1) * ch, ch)]],
                    bufs[o], sems[o])
            outs[b] = pltpu.async_copy(
                bufs[b], out_hbm.at[pl.ds(base + c * ch, ch)], osems[b])
        for b in range(2):
            if outs[b] is not None:
                outs[b].wait()

    return sc_gather


def kernel(feat, num_graphs):
    ntot, d = feat.shape
    n = ntot // B
    rb = 4096
    gb = rb // n

    keys = pl.pallas_call(
        _rowmax_kernel,
        grid=(ntot // rb,),
        in_specs=[pl.BlockSpec((rb, d), lambda i: (i, 0))],
        out_specs=pl.BlockSpec((gb, n), lambda i: (i, 0)),
        out_shape=jax.ShapeDtypeStruct((B, n), jnp.float32),
    )(feat)

    idx = pl.pallas_call(
        _topk_kernel,
        out_shape=jax.ShapeDtypeStruct((B, K), jnp.int32),
    )(keys)

    gathered = _make_sc_gather(ntot, d)(feat, idx.reshape(B * K))

    out = pl.pallas_call(
        _sort_kernel,
        grid=(B // G,),
        in_specs=[pl.BlockSpec((G * K, d), lambda s: (s, 0))],
        out_specs=pl.BlockSpec((G * K, d), lambda s: (s, 0)),
        out_shape=jax.ShapeDtypeStruct((B * K, d), jnp.float32),
    )(gathered)

    return out.reshape(B, K * d)


# speedup vs baseline: 27.1930x; 2.7425x over previous
import functools

import jax
import jax.numpy as jnp
from jax import lax
from jax.experimental import pallas as pl
from jax.experimental.pallas import tpu as pltpu
from jax.experimental.pallas import tpu_sc as plsc

B = 256
K = 64


def _rowmax_kernel(x_ref, o_ref):
    rb, d = x_ref.shape
    gb = o_ref.shape[0]
    o_ref[...] = jnp.max(x_ref[...].reshape(gb, rb // gb, d), axis=2)


def _topk_kernel(keys_ref, idx_ref):
    keys = keys_ref[...]
    nb, n = keys.shape
    iota = lax.broadcasted_iota(jnp.int32, (nb, n), 1)
    base = lax.broadcasted_iota(jnp.int32, (nb,), 0) * n
    neg = jnp.float32(-jnp.inf)
    for k in range(K):
        m = jnp.max(keys, axis=1, keepdims=True)
        cand = jnp.where(keys == m, iota, n)
        a = jnp.min(cand, axis=1)
        idx_ref[:, k] = base + a
        keys = jnp.where(iota == a[:, None], neg, keys)


def _sc_sort_row(vs):
    vs = [jnp.sort(v) for v in vs]
    r = 1
    while r < 16:
        out = []
        for p in range(0, 16, 2 * r):
            a = vs[p:p + r]
            b = vs[p + r:p + 2 * r]
            c = a + [lax.rev(x, (0,)) for x in reversed(b)]
            t = r
            while t >= 1:
                for i in range(2 * r):
                    if (i % (2 * t)) < t:
                        lo = jnp.minimum(c[i], c[i + t])
                        hi = jnp.maximum(c[i], c[i + t])
                        c[i], c[i + t] = lo, hi
                t //= 2
            out.extend(jnp.sort(x) for x in c)
        vs = out
        r *= 2
    return vs


def _make_sc_gather_sort(ntot, d):
    nsel = B * K
    nw = 32
    bpw = nsel // nw
    ch = 128
    nch = bpw // ch
    nv = d // 16
    mesh = plsc.VectorSubcoreMesh(core_axis_name="c", subcore_axis_name="s")

    @functools.partial(
        pl.kernel, mesh=mesh,
        compiler_params=pltpu.CompilerParams(needs_layout_passes=False),
        out_type=jax.ShapeDtypeStruct((nsel, d), jnp.float32),
        scratch_types=[
            pltpu.VMEM((bpw,), jnp.int32),
            pltpu.VMEM((ch, d), jnp.float32),
            pltpu.VMEM((ch, d), jnp.float32),
            pltpu.SemaphoreType.DMA,
            pltpu.SemaphoreType.DMA,
            pltpu.SemaphoreType.DMA,
            pltpu.SemaphoreType.DMA,
        ],
    )
    def sc_gather_sort(feat_hbm, idx_hbm, out_hbm, idx_v, buf0, buf1,
                       sem0, sem1, osem0, osem1):
        wid = lax.axis_index("s") * 2 + lax.axis_index("c")
        base = wid * bpw
        pltpu.sync_copy(idx_hbm.at[pl.ds(base, bpw)], idx_v)
        bufs, sems = (buf0, buf1), (sem0, sem1)
        osems = (osem0, osem1)
        gathers = [None, None]
        outs = [None, None]

        def sort_rows(buf):
            def body(row, _):
                vs = [buf[row, pl.ds(i * 16, 16)] for i in range(nv)]
                vs = _sc_sort_row(vs)
                for i in range(nv):
                    buf[row, pl.ds(i * 16, 16)] = vs[i]
                return _

            lax.fori_loop(0, ch, body, None)

        gathers[0] = pltpu.async_copy(
            feat_hbm.at[idx_v.at[pl.ds(0, ch)]], bufs[0], sems[0])
        for c in range(nch):
            b = c % 2
            o = (c + 1) % 2
            gathers[b].wait()
            if c + 1 < nch:
                if outs[o] is not None:
                    outs[o].wait()
                gathers[o] = pltpu.async_copy(
                    feat_hbm.at[idx_v.at[pl.ds((c + 1) * ch, ch)]],
                    bufs[o], sems[o])
            sort_rows(bufs[b])
            outs[b] = pltpu.async_copy(
                bufs[b], out_hbm.at[pl.ds(base + c * ch, ch)], osems[b])
        for b in range(2):
            if outs[b] is not None:
                outs[b].wait()

    return sc_gather_sort


def kernel(feat, num_graphs):
    ntot, d = feat.shape
    n = ntot // B
    rb = 4096
    gb = rb // n

    keys = pl.pallas_call(
        _rowmax_kernel,
        grid=(ntot // rb,),
        in_specs=[pl.BlockSpec((rb, d), lambda i: (i, 0))],
        out_specs=pl.BlockSpec((gb, n), lambda i: (i, 0)),
        out_shape=jax.ShapeDtypeStruct((B, n), jnp.float32),
    )(feat)

    idx = pl.pallas_call(
        _topk_kernel,
        out_shape=jax.ShapeDtypeStruct((B, K), jnp.int32),
    )(keys)

    out = _make_sc_gather_sort(ntot, d)(feat, idx.reshape(B * K))

    return out.reshape(B, K * d)


# speedup vs baseline: 27.6156x; 1.0155x over previous
import functools

import jax
import jax.numpy as jnp
from jax import lax
from jax.experimental import pallas as pl
from jax.experimental.pallas import tpu as pltpu
from jax.experimental.pallas import tpu_sc as plsc

B = 256
K = 64


def _select_kernel(x_ref, idx_ref, keys_ref):
    rb, d = x_ref.shape
    nb, n = keys_ref.shape
    gb = rb // n
    i = pl.program_id(0)
    keys_ref[pl.ds(i * gb, gb), :] = jnp.max(
        x_ref[...].reshape(gb, n, d), axis=2)

    @pl.when(i == pl.num_programs(0) - 1)
    def _():
        keys = keys_ref[...]
        iota = lax.broadcasted_iota(jnp.int32, (nb, n), 1)
        base = lax.broadcasted_iota(jnp.int32, (nb,), 0) * n
        neg = jnp.float32(-jnp.inf)
        for k in range(K):
            m = jnp.max(keys, axis=1, keepdims=True)
            cand = jnp.where(keys == m, iota, n)
            a = jnp.min(cand, axis=1)
            idx_ref[:, k] = base + a
            keys = jnp.where(iota == a[:, None], neg, keys)


def _sc_sort_row(vs):
    vs = [jnp.sort(v) for v in vs]
    r = 1
    while r < 16:
        out = []
        for p in range(0, 16, 2 * r):
            a = vs[p:p + r]
            b = vs[p + r:p + 2 * r]
            c = a + [lax.rev(x, (0,)) for x in reversed(b)]
            t = r
            while t >= 1:
                for i in range(2 * r):
                    if (i % (2 * t)) < t:
                        lo = jnp.minimum(c[i], c[i + t])
                        hi = jnp.maximum(c[i], c[i + t])
                        c[i], c[i + t] = lo, hi
                t //= 2
            out.extend(jnp.sort(x) for x in c)
        vs = out
        r *= 2
    return vs


def _make_sc_gather_sort(ntot, d):
    nsel = B * K
    nw = 32
    bpw = nsel // nw
    ch = 128
    nch = bpw // ch
    nv = d // 16
    mesh = plsc.VectorSubcoreMesh(core_axis_name="c", subcore_axis_name="s")

    @functools.partial(
        pl.kernel, mesh=mesh,
        compiler_params=pltpu.CompilerParams(needs_layout_passes=False),
        out_type=jax.ShapeDtypeStruct((nsel, d), jnp.float32),
        scratch_types=[
            pltpu.VMEM((bpw,), jnp.int32),
            pltpu.VMEM((ch, d), jnp.float32),
            pltpu.VMEM((ch, d), jnp.float32),
            pltpu.SemaphoreType.DMA,
            pltpu.SemaphoreType.DMA,
            pltpu.SemaphoreType.DMA,
            pltpu.SemaphoreType.DMA,
        ],
    )
    def sc_gather_sort(feat_hbm, idx_hbm, out_hbm, idx_v, buf0, buf1,
                       sem0, sem1, osem0, osem1):
        wid = lax.axis_index("s") * 2 + lax.axis_index("c")
        base = wid * bpw
        pltpu.sync_copy(idx_hbm.at[pl.ds(base, bpw)], idx_v)
        bufs, sems = (buf0, buf1), (sem0, sem1)
        osems = (osem0, osem1)
        gathers = [None, None]
        outs = [None, None]

        def sort_rows(buf):
            def body(row, _):
                vs = [buf[row, pl.ds(i * 16, 16)] for i in range(nv)]
                vs = _sc_sort_row(vs)
                for i in range(nv):
                    buf[row, pl.ds(i * 16, 16)] = vs[i]
                return _

            lax.fori_loop(0, ch, body, None)

        gathers[0] = pltpu.async_copy(
            feat_hbm.at[idx_v.at[pl.ds(0, ch)]], bufs[0], sems[0])
        for c in range(nch):
            b = c % 2
            o = (c + 1) % 2
            gathers[b].wait()
            if c + 1 < nch:
                if outs[o] is not None:
                    outs[o].wait()
                gathers[o] = pltpu.async_copy(
                    feat_hbm.at[idx_v.at[pl.ds((c + 1) * ch, ch)]],
                    bufs[o], sems[o])
            sort_rows(bufs[b])
            outs[b] = pltpu.async_copy(
                bufs[b], out_hbm.at[pl.ds(base + c * ch, ch)], osems[b])
        for b in range(2):
            if outs[b] is not None:
                outs[b].wait()

    return sc_gather_sort


def kernel(feat, num_graphs):
    ntot, d = feat.shape
    n = ntot // B
    rb = 4096

    idx = pl.pallas_call(
        _select_kernel,
        grid=(ntot // rb,),
        in_specs=[pl.BlockSpec((rb, d), lambda i: (i, 0))],
        out_specs=pl.BlockSpec((B, K), lambda i: (0, 0)),
        out_shape=jax.ShapeDtypeStruct((B, K), jnp.int32),
        scratch_shapes=[pltpu.VMEM((B, n), jnp.float32)],
    )(feat)

    out = _make_sc_gather_sort(ntot, d)(feat, idx.reshape(B * K))

    return out.reshape(B, K * d)


# speedup vs baseline: 28.3899x; 1.0280x over previous
import functools

import jax
import jax.numpy as jnp
from jax import lax
from jax.experimental import pallas as pl
from jax.experimental.pallas import tpu as pltpu
from jax.experimental.pallas import tpu_sc as plsc

B = 256
K = 64


def _select_kernel(x_ref, idx_ref, keys_ref):
    rb, d = x_ref.shape
    nb, n = keys_ref.shape
    gb = rb // n
    i = pl.program_id(0)
    keys_ref[pl.ds(i * gb, gb), :] = jnp.max(
        x_ref[...].reshape(gb, n, d), axis=2)

    @pl.when(i == pl.num_programs(0) - 1)
    def _():
        keys = keys_ref[...]
        iota = lax.broadcasted_iota(jnp.int32, (nb, n), 1)
        base = lax.broadcasted_iota(jnp.int32, (nb,), 0) * n
        neg = jnp.float32(-jnp.inf)
        for k in range(K):
            m = jnp.max(keys, axis=1, keepdims=True)
            cand = jnp.where(keys == m, iota, n)
            a = jnp.min(cand, axis=1)
            idx_ref[:, k] = base + a
            keys = jnp.where(iota == a[:, None], neg, keys)


def _vsort(v, asc):
    if asc:
        return jnp.sort(v)
    r = plsc.sort_key_val(v, v, descending=True)
    return r[0] if isinstance(r, (tuple, list)) else r


def _sc_sort_row(vs, asc=True):
    if len(vs) == 1:
        return [_vsort(vs[0], asc)]
    h = len(vs) // 2
    c = _sc_sort_row(vs[:h], True) + _sc_sort_row(vs[h:], False)
    t = h
    while t >= 1:
        for i in range(len(c)):
            if (i % (2 * t)) < t:
                lo = jnp.minimum(c[i], c[i + t])
                hi = jnp.maximum(c[i], c[i + t])
                if asc:
                    c[i], c[i + t] = lo, hi
                else:
                    c[i], c[i + t] = hi, lo
        t //= 2
    return [_vsort(x, asc) for x in c]


def _make_sc_gather_sort(ntot, d):
    nsel = B * K
    nw = 32
    bpw = nsel // nw
    ch = 128
    nch = bpw // ch
    nv = d // 16
    mesh = plsc.VectorSubcoreMesh(core_axis_name="c", subcore_axis_name="s")

    @functools.partial(
        pl.kernel, mesh=mesh,
        compiler_params=pltpu.CompilerParams(needs_layout_passes=False),
        out_type=jax.ShapeDtypeStruct((nsel, d), jnp.float32),
        scratch_types=[
            pltpu.VMEM((bpw,), jnp.int32),
            pltpu.VMEM((ch, d), jnp.float32),
            pltpu.VMEM((ch, d), jnp.float32),
            pltpu.SemaphoreType.DMA,
            pltpu.SemaphoreType.DMA,
            pltpu.SemaphoreType.DMA,
            pltpu.SemaphoreType.DMA,
        ],
    )
    def sc_gather_sort(feat_hbm, idx_hbm, out_hbm, idx_v, buf0, buf1,
                       sem0, sem1, osem0, osem1):
        wid = lax.axis_index("s") * 2 + lax.axis_index("c")
        base = wid * bpw
        pltpu.sync_copy(idx_hbm.at[pl.ds(base, bpw)], idx_v)
        bufs, sems = (buf0, buf1), (sem0, sem1)
        osems = (osem0, osem1)
        gathers = [None, None]
        outs = [None, None]

        def sort_rows(buf):
            def body(row, _):
                vs = [buf[row, pl.ds(i * 16, 16)] for i in range(nv)]
                vs = _sc_sort_row(vs)
                for i in range(nv):
                    buf[row, pl.ds(i * 16, 16)] = vs[i]
                return _

            lax.fori_loop(0, ch, body, None)

        gathers[0] = pltpu.async_copy(
            feat_hbm.at[idx_v.at[pl.ds(0, ch)]], bufs[0], sems[0])
        for c in range(nch):
            b = c % 2
            o = (c + 1) % 2
            gathers[b].wait()
            if c + 1 < nch:
                if outs[o] is not None:
                    outs[o].wait()
                gathers[o] = pltpu.async_copy(
                    feat_hbm.at[idx_v.at[pl.ds((c + 1) * ch, ch)]],
                    bufs[o], sems[o])
            sort_rows(bufs[b])
            outs[b] = pltpu.async_copy(
                bufs[b], out_hbm.at[pl.ds(base + c * ch, ch)], osems[b])
        for b in range(2):
            if outs[b] is not None:
                outs[b].wait()

    return sc_gather_sort


def kernel(feat, num_graphs):
    ntot, d = feat.shape
    n = ntot // B
    rb = 4096

    idx = pl.pallas_call(
        _select_kernel,
        grid=(ntot // rb,),
        in_specs=[pl.BlockSpec((rb, d), lambda i: (i, 0))],
        out_specs=pl.BlockSpec((B, K), lambda i: (0, 0)),
        out_shape=jax.ShapeDtypeStruct((B, K), jnp.int32),
        scratch_shapes=[pltpu.VMEM((B, n), jnp.float32)],
    )(feat)

    out = _make_sc_gather_sort(ntot, d)(feat, idx.reshape(B * K))

    return out.reshape(B, K * d)


# speedup vs baseline: 29.4731x; 1.0382x over previous
import functools

import jax
import jax.numpy as jnp
from jax import lax
from jax.experimental import pallas as pl
from jax.experimental.pallas import tpu as pltpu
from jax.experimental.pallas import tpu_sc as plsc

B = 256
K = 64


def _select_kernel(x_ref, idx_ref, keys_ref):
    rb, d = x_ref.shape
    nb, n = keys_ref.shape
    gb = rb // n
    i = pl.program_id(0)
    keys_ref[pl.ds(i * gb, gb), :] = jnp.max(
        x_ref[...].reshape(gb, n, d), axis=2)

    @pl.when(i == pl.num_programs(0) - 1)
    def _():
        keys = keys_ref[...]
        iota = lax.broadcasted_iota(jnp.int32, (nb, n), 1)
        base = lax.broadcasted_iota(jnp.int32, (nb,), 0) * n
        neg = jnp.float32(-jnp.inf)
        for k in range(K):
            m = jnp.max(keys, axis=1, keepdims=True)
            cand = jnp.where(keys == m, iota, n)
            a = jnp.min(cand, axis=1)
            idx_ref[:, k] = base + a
            keys = jnp.where(iota == a[:, None], neg, keys)


def _vsort(v, asc):
    if asc:
        return jnp.sort(v)
    r = plsc.sort_key_val(v, v, descending=True)
    return r[0] if isinstance(r, (tuple, list)) else r


def _sc_sort_row(vs, asc=True):
    if len(vs) == 1:
        return [_vsort(vs[0], asc)]
    h = len(vs) // 2
    c = _sc_sort_row(vs[:h], True) + _sc_sort_row(vs[h:], False)
    t = h
    while t >= 1:
        for i in range(len(c)):
            if (i % (2 * t)) < t:
                lo = jnp.minimum(c[i], c[i + t])
                hi = jnp.maximum(c[i], c[i + t])
                if asc:
                    c[i], c[i + t] = lo, hi
                else:
                    c[i], c[i + t] = hi, lo
        t //= 2
    return [_vsort(x, asc) for x in c]


def _make_sc_gather_sort(ntot, d):
    nsel = B * K
    nw = 32
    bpw = nsel // nw
    ch = 64
    nch = bpw // ch
    nv = d // 16
    mesh = plsc.VectorSubcoreMesh(core_axis_name="c", subcore_axis_name="s")

    @functools.partial(
        pl.kernel, mesh=mesh,
        compiler_params=pltpu.CompilerParams(needs_layout_passes=False),
        out_type=jax.ShapeDtypeStruct((nsel, d), jnp.float32),
        scratch_types=[
            pltpu.VMEM((bpw,), jnp.int32),
            pltpu.VMEM((ch, d), jnp.float32),
            pltpu.VMEM((ch, d), jnp.float32),
            pltpu.SemaphoreType.DMA,
            pltpu.SemaphoreType.DMA,
            pltpu.SemaphoreType.DMA,
            pltpu.SemaphoreType.DMA,
        ],
    )
    def sc_gather_sort(feat_hbm, idx_hbm, out_hbm, idx_v, buf0, buf1,
                       sem0, sem1, osem0, osem1):
        wid = lax.axis_index("s") * 2 + lax.axis_index("c")
        base = wid * bpw
        pltpu.sync_copy(idx_hbm.at[pl.ds(base, bpw)], idx_v)
        bufs, sems = (buf0, buf1), (sem0, sem1)
        osems = (osem0, osem1)
        gathers = [None, None]
        outs = [None, None]

        def sort_rows(buf):
            def body(row, _):
                vs = [buf[row, pl.ds(i * 16, 16)] for i in range(nv)]
                vs = _sc_sort_row(vs)
                for i in range(nv):
                    buf[row, pl.ds(i * 16, 16)] = vs[i]
                return _

            lax.fori_loop(0, ch, body, None)

        gathers[0] = pltpu.async_copy(
            feat_hbm.at[idx_v.at[pl.ds(0, ch)]], bufs[0], sems[0])
        for c in range(nch):
            b = c % 2
            o = (c + 1) % 2
            gathers[b].wait()
            if c + 1 < nch:
                if outs[o] is not None:
                    outs[o].wait()
                gathers[o] = pltpu.async_copy(
                    feat_hbm.at[idx_v.at[pl.ds((c + 1) * ch, ch)]],
                    bufs[o], sems[o])
            sort_rows(bufs[b])
            outs[b] = pltpu.async_copy(
                bufs[b], out_hbm.at[pl.ds(base + c * ch, ch)], osems[b])
        for b in range(2):
            if outs[b] is not None:
                outs[b].wait()

    return sc_gather_sort


def kernel(feat, num_graphs):
    ntot, d = feat.shape
    n = ntot // B
    rb = 8192

    idx = pl.pallas_call(
        _select_kernel,
        grid=(ntot // rb,),
        in_specs=[pl.BlockSpec((rb, d), lambda i: (i, 0))],
        out_specs=pl.BlockSpec((B, K), lambda i: (0, 0)),
        out_shape=jax.ShapeDtypeStruct((B, K), jnp.int32),
        scratch_shapes=[pltpu.VMEM((B, n), jnp.float32)],
    )(feat)

    out = _make_sc_gather_sort(ntot, d)(feat, idx.reshape(B * K))

    return out.reshape(B, K * d)
